# baseline (speedup 1.0000x reference)
import functools

import jax
import jax.numpy as jnp
from jax import lax
from jax.experimental import pallas as pl
from jax.experimental.pallas import tpu as pltpu
from jax.experimental.pallas import tpu_sc as plsc

_GRID = 64
_NVOX = _GRID ** 3
_NWORDS = _NVOX // 32
_VOXEL_SIZE = 0.1
_OFFSET = 3.2
_SCENE_C = 16
_VFREQS = 6
_XFREQS = 10
_OUT_D = _SCENE_C * (1 + 2 * _VFREQS) + 3 * (1 + 2 * _XFREQS)

_NW = 32
_T = 512
_G = _T // 16
_P4 = 4 * _T


def _srl(v, k):
    return lax.shift_right_logical(v, jnp.broadcast_to(jnp.int32(k), v.shape))


def _popc(v):
    v = v - (_srl(v, 1) & 0x55555555)
    v = (v & 0x33333333) + (_srl(v, 2) & 0x33333333)
    v = (v + _srl(v, 4)) & 0x0F0F0F0F
    return _srl(v * 0x01010101, 24)


def _floor16(s):
    q = s.astype(jnp.int32)
    qf = q.astype(jnp.float32)
    adj = qf > s
    q = jnp.where(adj, q - 1, q)
    qf = jnp.where(adj, qf - 1.0, qf)
    return q, s - qf


def _sc_body(npw, xs, ys, zs, bits_h, pref_h, tpair, scene,
             xbuf, ybuf, zbuf, bitsb, prefb, pidx, w0b, w1b, rows, acc,
             sem, gsem):
    wid = lax.axis_index("s") * 2 + lax.axis_index("c")
    base0 = wid * npw

    pltpu.sync_copy(bits_h, bitsb)
    pltpu.sync_copy(pref_h, prefb)

    def lookup(f):
        wd = _srl(f, 5)
        b = f & 31
        bits = plsc.load_gather(bitsb, [wd])
        pref = plsc.load_gather(prefb, [wd])
        occ = (lax.shift_right_logical(bits, b) & 1) != 0
        below = bits & (lax.shift_left(jnp.full(f.shape, 1, jnp.int32), b) - 1)
        emb = pref + _popc(below)
        return occ, emb

    def tile(t, _):
        base = base0 + t * _T
        pltpu.sync_copy(xs.at[pl.ds(base, _T)], xbuf)
        pltpu.sync_copy(ys.at[pl.ds(base, _T)], ybuf)
        pltpu.sync_copy(zs.at[pl.ds(base, _T)], zbuf)

        @plsc.parallel_loop(0, _G)
        def passa(g):
            sl = pl.ds(g * 16, 16)
            sx = (xbuf[sl] + _OFFSET) / _VOXEL_SIZE
            sy = (ybuf[sl] + _OFFSET) / _VOXEL_SIZE
            sz = (zbuf[sl] + _OFFSET) / _VOXEL_SIZE
            qx, u = _floor16(sx)
            qy, v = _floor16(sy)
            qz, w = _floor16(sz)
            lu, lv, lw = 1.0 - u, 1.0 - v, 1.0 - w
            vx = ((qx >= 0) & (qx < _GRID), (qx >= -1) & (qx < _GRID - 1))
            vy = ((qy >= 0) & (qy < _GRID), (qy >= -1) & (qy < _GRID - 1))
            vz0 = (qz >= 0) & (qz < _GRID)
            vz1 = (qz >= -1) & (qz < _GRID - 1)
            zero = jnp.zeros((16,), jnp.int32)
            for p, (a, b) in enumerate(((0, 0), (0, 1), (1, 0), (1, 1))):
                gv = vx[a] & vy[b]
                f0 = ((qx + a) * _GRID + (qy + b)) * _GRID + qz
                g0 = gv & vz0
                g1 = gv & vz1
                occ0, emb0 = lookup(jnp.where(g0, f0, zero))
                occ1, emb1 = lookup(jnp.where(g1, f0 + 1, zero))
                occ0 = occ0 & g0
                occ1 = occ1 & g1
                wa = u if a else lu
                wb = v if b else lv
                wgt0 = wa * wb * lw
                wgt1 = wa * wb * w
                dst = pl.ds(p * _T + g * 16, 16)
                pidx[dst] = jnp.where(occ0, emb0, jnp.where(occ1, emb1, zero))
                w0b[dst] = jnp.where(occ0, wgt0, jnp.where(occ1, wgt1, 0.0))
                w1b[dst] = jnp.where(occ0 & occ1, wgt1, 0.0)

        pltpu.make_async_copy(tpair.at[pidx], rows, gsem).start()
        pltpu.make_async_copy(tpair.at[pidx], rows, gsem).wait()

        @plsc.parallel_loop(0, _G)
        def blend(g):
            w0v = [w0b[pl.ds(p * _T + g * 16, 16)] for p in range(4)]
            w1v = [w1b[pl.ds(p * _T + g * 16, 16)] for p in range(4)]
            for j in range(16):
                i = g * 16 + j
                av = w0v[0][j] * rows[0 * _T + i, 0:16]
                av = av + w1v[0][j] * rows[0 * _T + i, 16:32]
                for p in range(1, 4):
                    av = av + w0v[p][j] * rows[p * _T + i, 0:16]
                    av = av + w1v[p][j] * rows[p * _T + i, 16:32]
                acc[i, :] = av

        pltpu.sync_copy(acc, scene.at[pl.ds(base, _T)])
        return 0

    lax.fori_loop(0, npw // _T, tile, 0)


def _tc_body(scene_ref, xyz_ref, out_ref):
    s = scene_ref[...]
    x = xyz_ref[...]
    parts = [s]
    f = 1.0
    for _ in range(_VFREQS):
        parts.append(jnp.sin(f * s))
        parts.append(jnp.cos(f * s))
        f *= 2.0
    parts.append(x)
    f = 1.0
    for _ in range(_XFREQS):
        parts.append(jnp.sin(f * x))
        parts.append(jnp.cos(f * x))
        f *= 2.0
    out_ref[...] = jnp.concatenate(parts, axis=-1)


@jax.jit
def kernel(xyz, table, voxel_idx_map):
    n, sample, _ = xyz.shape
    npts = n * sample
    npw = npts // _NW
    x = xyz.reshape(-1, 3)
    xs, ys, zs = x[:, 0], x[:, 1], x[:, 2]

    flat = voxel_idx_map.reshape(-1) >= 0
    wordsb = flat.reshape(_NWORDS, 32).astype(jnp.uint32)
    shifts = jnp.left_shift(jnp.uint32(1), jnp.arange(32, dtype=jnp.uint32))
    bits = lax.bitcast_convert_type((wordsb * shifts).sum(axis=1,
                                                          dtype=jnp.uint32),
                                    jnp.int32)
    counts = wordsb.sum(axis=1, dtype=jnp.int32)
    pref = jnp.concatenate([jnp.zeros((1,), jnp.int32),
                            jnp.cumsum(counts)[:-1].astype(jnp.int32)])

    t16 = table[:, :_SCENE_C]
    tpair = jnp.concatenate(
        [t16, jnp.concatenate([t16[1:], jnp.zeros((1, _SCENE_C),
                                                  jnp.float32)])], axis=1)

    mesh = plsc.VectorSubcoreMesh(core_axis_name="c", subcore_axis_name="s")
    scene = pl.kernel(
        functools.partial(_sc_body, npw),
        out_type=jax.ShapeDtypeStruct((npts, _SCENE_C), jnp.float32),
        mesh=mesh,
        compiler_params=pltpu.CompilerParams(use_tc_tiling_on_sc=False,
                                             needs_layout_passes=False),
        scratch_types=[
            pltpu.VMEM((_T,), jnp.float32),
            pltpu.VMEM((_T,), jnp.float32),
            pltpu.VMEM((_T,), jnp.float32),
            pltpu.VMEM((_NWORDS,), jnp.int32),
            pltpu.VMEM((_NWORDS,), jnp.int32),
            pltpu.VMEM((_P4,), jnp.int32),
            pltpu.VMEM((_P4,), jnp.float32),
            pltpu.VMEM((_P4,), jnp.float32),
            pltpu.VMEM((_P4, 2 * _SCENE_C), jnp.float32),
            pltpu.VMEM((_T, _SCENE_C), jnp.float32),
            pltpu.SemaphoreType.DMA,
            pltpu.SemaphoreType.DMA,
        ],
    )(xs, ys, zs, bits, pref, tpair)

    blk = 2048
    out = pl.pallas_call(
        _tc_body,
        grid=(npts // blk,),
        in_specs=[
            pl.BlockSpec((blk, _SCENE_C), lambda i: (i, 0)),
            pl.BlockSpec((blk, 3), lambda i: (i, 0)),
        ],
        out_specs=pl.BlockSpec((blk, _OUT_D), lambda i: (i, 0)),
        out_shape=jax.ShapeDtypeStruct((npts, _OUT_D), jnp.float32),
    )(scene, x)
    return out.reshape(n, sample, _OUT_D)


# speedup vs baseline: 1.8853x; 1.8853x over previous
import functools

import jax
import jax.numpy as jnp
from jax import lax
from jax.experimental import pallas as pl
from jax.experimental.pallas import tpu as pltpu
from jax.experimental.pallas import tpu_sc as plsc

_GRID = 64
_NVOX = _GRID ** 3
_NWORDS = _NVOX // 32
_VOXEL_SIZE = 0.1
_OFFSET = 3.2
_SCENE_C = 16
_VFREQS = 6
_XFREQS = 10
_OUT_D = _SCENE_C * (1 + 2 * _VFREQS) + 3 * (1 + 2 * _XFREQS)

_NW = 32
_T = 512
_G = _T // 16
_P4 = 4 * _T


def _srl(v, k):
    return lax.shift_right_logical(v, jnp.broadcast_to(jnp.int32(k), v.shape))


def _popc(v):
    v = v - (_srl(v, 1) & 0x55555555)
    v = (v & 0x33333333) + (_srl(v, 2) & 0x33333333)
    v = (v + _srl(v, 4)) & 0x0F0F0F0F
    return _srl(v * 0x01010101, 24)


def _floor16(s):
    q = s.astype(jnp.int32)
    qf = q.astype(jnp.float32)
    adj = qf > s
    q = jnp.where(adj, q - 1, q)
    qf = jnp.where(adj, qf - 1.0, qf)
    return q, s - qf


def _sc_body(npw, xs, ys, zs, bits_h, pref_h, tpair, scene,
             xbuf, ybuf, zbuf, bitsb, prefb, pidx, w0b, w1b, rows, acc,
             sem, gsem):
    wid = lax.axis_index("s") * 2 + lax.axis_index("c")
    base0 = wid * npw

    pltpu.sync_copy(bits_h, bitsb)
    pltpu.sync_copy(pref_h, prefb)

    def lookup(f):
        wd = _srl(f, 5)
        b = f & 31
        bits = plsc.load_gather(bitsb, [wd])
        pref = plsc.load_gather(prefb, [wd])
        occ = (lax.shift_right_logical(bits, b) & 1) != 0
        below = bits & (lax.shift_left(jnp.full(f.shape, 1, jnp.int32), b) - 1)
        emb = pref + _popc(below)
        return occ, emb

    def tile(t, _):
        base = base0 + t * _T
        pltpu.sync_copy(xs.at[pl.ds(base, _T)], xbuf)
        pltpu.sync_copy(ys.at[pl.ds(base, _T)], ybuf)
        pltpu.sync_copy(zs.at[pl.ds(base, _T)], zbuf)

        @plsc.parallel_loop(0, _G)
        def passa(g):
            sl = pl.ds(g * 16, 16)
            sx = (xbuf[sl] + _OFFSET) / _VOXEL_SIZE
            sy = (ybuf[sl] + _OFFSET) / _VOXEL_SIZE
            sz = (zbuf[sl] + _OFFSET) / _VOXEL_SIZE
            qx, u = _floor16(sx)
            qy, v = _floor16(sy)
            qz, w = _floor16(sz)
            lu, lv, lw = 1.0 - u, 1.0 - v, 1.0 - w
            vx = ((qx >= 0) & (qx < _GRID), (qx >= -1) & (qx < _GRID - 1))
            vy = ((qy >= 0) & (qy < _GRID), (qy >= -1) & (qy < _GRID - 1))
            vz0 = (qz >= 0) & (qz < _GRID)
            vz1 = (qz >= -1) & (qz < _GRID - 1)
            zero = jnp.zeros((16,), jnp.int32)
            for p, (a, b) in enumerate(((0, 0), (0, 1), (1, 0), (1, 1))):
                gv = vx[a] & vy[b]
                f0 = ((qx + a) * _GRID + (qy + b)) * _GRID + qz
                g0 = gv & vz0
                g1 = gv & vz1
                occ0, emb0 = lookup(jnp.where(g0, f0, zero))
                occ1, emb1 = lookup(jnp.where(g1, f0 + 1, zero))
                occ0 = occ0 & g0
                occ1 = occ1 & g1
                wa = u if a else lu
                wb = v if b else lv
                wgt0 = wa * wb * lw
                wgt1 = wa * wb * w
                dst = pl.ds(p * _T + g * 16, 16)
                pidx[dst] = jnp.where(occ0, emb0, jnp.where(occ1, emb1, zero))
                w0b[dst] = jnp.where(occ0, wgt0, jnp.where(occ1, wgt1, 0.0))
                w1b[dst] = jnp.where(occ0 & occ1, wgt1, 0.0)

        pltpu.make_async_copy(tpair.at[pidx], rows, gsem).start()
        pltpu.make_async_copy(tpair.at[pidx], rows, gsem).wait()

        @plsc.parallel_loop(0, _G)
        def blend(g):
            w0v = [w0b[pl.ds(p * _T + g * 16, 16)] for p in range(4)]
            w1v = [w1b[pl.ds(p * _T + g * 16, 16)] for p in range(4)]
            for j in range(16):
                i = g * 16 + j
                av = w0v[0][j] * rows[0 * _T + i, 0:16]
                av = av + w1v[0][j] * rows[0 * _T + i, 16:32]
                for p in range(1, 4):
                    av = av + w0v[p][j] * rows[p * _T + i, 0:16]
                    av = av + w1v[p][j] * rows[p * _T + i, 16:32]
                acc[i, :] = av

        pltpu.sync_copy(acc, scene.at[pl.ds(base, _T)])
        return 0

    lax.fori_loop(0, npw // _T, tile, 0)


def _masked_embed(rep, width):
    p = lax.broadcasted_iota(jnp.int32, rep.shape, 1) // width
    j = jnp.maximum(p - 1, 0)
    scale = lax.bitcast_convert_type((j // 2 + 127) << 23, jnp.float32)
    t = rep * scale
    return jnp.where(p == 0, rep,
                     jnp.where(j % 2 == 0, jnp.sin(t), jnp.cos(t)))


def _tc_body(scene_ref, xyz_ref, out_ref):
    s = scene_ref[...]
    x = xyz_ref[...]
    vox = _masked_embed(jnp.tile(s, (1, 1 + 2 * _VFREQS)), _SCENE_C)
    xyzp = _masked_embed(jnp.tile(x, (1, 1 + 2 * _XFREQS)), 3)
    out_ref[...] = jnp.concatenate([vox, xyzp], axis=-1)


@jax.jit
def kernel(xyz, table, voxel_idx_map):
    n, sample, _ = xyz.shape
    npts = n * sample
    npw = npts // _NW
    x = xyz.reshape(-1, 3)
    xs, ys, zs = x[:, 0], x[:, 1], x[:, 2]

    flat = voxel_idx_map.reshape(-1) >= 0
    wordsb = flat.reshape(_NWORDS, 32).astype(jnp.uint32)
    shifts = jnp.left_shift(jnp.uint32(1), jnp.arange(32, dtype=jnp.uint32))
    bits = lax.bitcast_convert_type((wordsb * shifts).sum(axis=1,
                                                          dtype=jnp.uint32),
                                    jnp.int32)
    counts = wordsb.sum(axis=1, dtype=jnp.int32)
    pref = jnp.concatenate([jnp.zeros((1,), jnp.int32),
                            jnp.cumsum(counts)[:-1].astype(jnp.int32)])

    t16 = table[:, :_SCENE_C]
    tpair = jnp.concatenate(
        [t16, jnp.concatenate([t16[1:], jnp.zeros((1, _SCENE_C),
                                                  jnp.float32)])], axis=1)

    mesh = plsc.VectorSubcoreMesh(core_axis_name="c", subcore_axis_name="s")
    scene = pl.kernel(
        functools.partial(_sc_body, npw),
        out_type=jax.ShapeDtypeStruct((npts, _SCENE_C), jnp.float32),
        mesh=mesh,
        compiler_params=pltpu.CompilerParams(use_tc_tiling_on_sc=False,
                                             needs_layout_passes=False),
        scratch_types=[
            pltpu.VMEM((_T,), jnp.float32),
            pltpu.VMEM((_T,), jnp.float32),
            pltpu.VMEM((_T,), jnp.float32),
            pltpu.VMEM((_NWORDS,), jnp.int32),
            pltpu.VMEM((_NWORDS,), jnp.int32),
            pltpu.VMEM((_P4,), jnp.int32),
            pltpu.VMEM((_P4,), jnp.float32),
            pltpu.VMEM((_P4,), jnp.float32),
            pltpu.VMEM((_P4, 2 * _SCENE_C), jnp.float32),
            pltpu.VMEM((_T, _SCENE_C), jnp.float32),
            pltpu.SemaphoreType.DMA,
            pltpu.SemaphoreType.DMA,
        ],
    )(xs, ys, zs, bits, pref, tpair)

    blk = 2048
    out = pl.pallas_call(
        _tc_body,
        grid=(npts // blk,),
        in_specs=[
            pl.BlockSpec((blk, _SCENE_C), lambda i: (i, 0)),
            pl.BlockSpec((blk, 3), lambda i: (i, 0)),
        ],
        out_specs=pl.BlockSpec((blk, _OUT_D), lambda i: (i, 0)),
        out_shape=jax.ShapeDtypeStruct((npts, _OUT_D), jnp.float32),
    )(scene, x)
    return out.reshape(n, sample, _OUT_D)
